# baseline (speedup 1.0000x reference)
import functools

import jax
import jax.numpy as jnp
from jax import lax
from jax.experimental import pallas as pl
from jax.experimental.pallas import tpu as pltpu
from jax.experimental.pallas import tpu_sc as plsc

BATCH = 16384
EMBED_DIM = 64
NUM_WORKERS = 32
B_PER_W = BATCH // NUM_WORKERS
GROUP = 2
GROUPS = B_PER_W // GROUP
IDX_PAD = 16

_mesh = plsc.VectorSubcoreMesh(core_axis_name="c", subcore_axis_name="s")


@functools.partial(
    pl.kernel,
    mesh=_mesh,
    compiler_params=pltpu.CompilerParams(needs_layout_passes=False),
    out_type=jax.ShapeDtypeStruct((BATCH,), jnp.float32),
    scratch_types=[
        pltpu.VMEM((B_PER_W + IDX_PAD,), jnp.int32),
        pltpu.VMEM((B_PER_W + IDX_PAD,), jnp.int32),
        pltpu.VMEM((2, GROUP, EMBED_DIM, 128), jnp.float32),
        pltpu.VMEM((2, GROUP, EMBED_DIM, 128), jnp.float32),
        pltpu.VMEM((B_PER_W,), jnp.float32),
        pltpu.SemaphoreType.DMA,
        pltpu.SemaphoreType.DMA,
        pltpu.SemaphoreType.DMA,
        pltpu.SemaphoreType.DMA,
    ],
)
def _dot_kernel(uid_hbm, iid_hbm, utT_hbm, itT_hbm, out_hbm,
                uidx_v, iidx_v, uslab_v, islab_v, out_v,
                sem_u0, sem_i0, sem_u1, sem_i1):
    wid = lax.axis_index("s") * 2 + lax.axis_index("c")
    base = wid * B_PER_W
    sems = ((sem_u0, sem_i0), (sem_u1, sem_i1))

    pltpu.sync_copy(uid_hbm.at[pl.ds(base, B_PER_W)],
                    uidx_v.at[pl.ds(0, B_PER_W)])
    pltpu.sync_copy(iid_hbm.at[pl.ds(base, B_PER_W)],
                    iidx_v.at[pl.ds(0, B_PER_W)])

    zeros16 = jnp.zeros((16,), jnp.float32)

    def zero_body(i, carry):
        out_v[pl.ds(i * 16, 16)] = zeros16
        return carry
    lax.fori_loop(0, B_PER_W // 16, zero_body, 0)

    lane = lax.iota(jnp.int32, 16)
    row_of_lane = jnp.bitwise_and(lane, GROUP - 1)
    qblk = lax.shift_right_logical(lane, 1)

    def issue(phase, g):
        uvec = uidx_v[pl.ds(g * GROUP, 16)]
        ivec = iidx_v[pl.ds(g * GROUP, 16)]
        for l in range(GROUP):
            ua = pl.multiple_of(
                lax.shift_left(lax.shift_right_logical(uvec[l], 7), 7), 128)
            ia = pl.multiple_of(
                lax.shift_left(lax.shift_right_logical(ivec[l], 7), 7), 128)
            pltpu.async_copy(utT_hbm.at[:, pl.ds(ua, 128)],
                             uslab_v.at[phase, l], sems[phase][0])
            pltpu.async_copy(itT_hbm.at[:, pl.ds(ia, 128)],
                             islab_v.at[phase, l], sems[phase][1])

    def wait_phase(phase):
        for l in range(GROUP):
            pltpu.make_async_copy(utT_hbm.at[:, pl.ds(0, 128)],
                                  uslab_v.at[phase, l],
                                  sems[phase][0]).wait()
            pltpu.make_async_copy(itT_hbm.at[:, pl.ds(0, 128)],
                                  islab_v.at[phase, l],
                                  sems[phase][1]).wait()

    def compute(phase, g):
        wait_phase(phase)
        um16 = jnp.bitwise_and(
            plsc.load_gather(uidx_v, [g * GROUP + row_of_lane]), 127)
        im16 = jnp.bitwise_and(
            plsc.load_gather(iidx_v, [g * GROUP + row_of_lane]), 127)
        acc = jnp.zeros((16,), jnp.float32)
        for j in range(EMBED_DIM // 8):
            dv = lax.shift_left(qblk, 3) + j
            uu = plsc.load_gather(uslab_v.at[phase], [row_of_lane, dv, um16])
            ii = plsc.load_gather(islab_v.at[phase], [row_of_lane, dv, im16])
            acc = acc + uu * ii
        plsc.addupdate_scatter(out_v, [g * GROUP + row_of_lane], acc)

    def body(p, carry):
        g0 = p * 2
        issue(1, g0 + 1)
        compute(0, g0)
        issue(0, jnp.minimum(g0 + 2, GROUPS - 2))
        compute(1, g0 + 1)
        return carry

    issue(0, 0)
    lax.fori_loop(0, GROUPS // 2, body, 0)
    wait_phase(0)

    pltpu.sync_copy(out_v, out_hbm.at[pl.ds(base, B_PER_W)])


def kernel(inputs, user_table, item_table):
    user_ids = inputs[:, 0].astype(jnp.int32)
    item_ids = inputs[:, 1].astype(jnp.int32)
    return _dot_kernel(user_ids, item_ids, user_table.T, item_table.T)


# speedup vs baseline: 1.1053x; 1.1053x over previous
import functools

import jax
import jax.numpy as jnp
from jax import lax
from jax.experimental import pallas as pl
from jax.experimental.pallas import tpu as pltpu
from jax.experimental.pallas import tpu_sc as plsc

BATCH = 16384
EMBED_DIM = 64
NUM_WORKERS = 32
B_PER_W = BATCH // NUM_WORKERS
GROUP = 2
GROUPS = B_PER_W // GROUP
IDX_PAD = 16

_mesh = plsc.VectorSubcoreMesh(core_axis_name="c", subcore_axis_name="s")


@functools.partial(
    pl.kernel,
    mesh=_mesh,
    compiler_params=pltpu.CompilerParams(needs_layout_passes=False),
    out_type=jax.ShapeDtypeStruct((BATCH,), jnp.float32),
    scratch_types=[
        pltpu.VMEM((B_PER_W + IDX_PAD,), jnp.int32),
        pltpu.VMEM((B_PER_W + IDX_PAD,), jnp.int32),
        pltpu.VMEM((3, GROUP, EMBED_DIM, 128), jnp.float32),
        pltpu.VMEM((3, GROUP, EMBED_DIM, 128), jnp.float32),
        pltpu.VMEM((B_PER_W,), jnp.float32),
        pltpu.SemaphoreType.DMA,
        pltpu.SemaphoreType.DMA,
        pltpu.SemaphoreType.DMA,
        pltpu.SemaphoreType.DMA,
        pltpu.SemaphoreType.DMA,
        pltpu.SemaphoreType.DMA,
    ],
)
def _dot_kernel(uid_hbm, iid_hbm, utT_hbm, itT_hbm, out_hbm,
                uidx_v, iidx_v, uslab_v, islab_v, out_v,
                sem_u0, sem_i0, sem_u1, sem_i1, sem_u2, sem_i2):
    wid = lax.axis_index("s") * 2 + lax.axis_index("c")
    base = wid * B_PER_W
    sems = ((sem_u0, sem_i0), (sem_u1, sem_i1), (sem_u2, sem_i2))

    pltpu.sync_copy(uid_hbm.at[pl.ds(base, B_PER_W)],
                    uidx_v.at[pl.ds(0, B_PER_W)])
    pltpu.sync_copy(iid_hbm.at[pl.ds(base, B_PER_W)],
                    iidx_v.at[pl.ds(0, B_PER_W)])

    zeros16 = jnp.zeros((16,), jnp.float32)

    def zero_body(i, carry):
        out_v[pl.ds(i * 16, 16)] = zeros16
        return carry
    lax.fori_loop(0, B_PER_W // 16, zero_body, 0)

    lane = lax.iota(jnp.int32, 16)
    row_of_lane = jnp.bitwise_and(lane, GROUP - 1)
    qblk = lax.shift_right_logical(lane, 1)

    def issue(phase, g):
        uvec = uidx_v[pl.ds(g * GROUP, 16)]
        ivec = iidx_v[pl.ds(g * GROUP, 16)]
        for l in range(GROUP):
            ua = pl.multiple_of(
                lax.shift_left(lax.shift_right_logical(uvec[l], 7), 7), 128)
            ia = pl.multiple_of(
                lax.shift_left(lax.shift_right_logical(ivec[l], 7), 7), 128)
            pltpu.async_copy(utT_hbm.at[:, pl.ds(ua, 128)],
                             uslab_v.at[phase, l], sems[phase][0])
            pltpu.async_copy(itT_hbm.at[:, pl.ds(ia, 128)],
                             islab_v.at[phase, l], sems[phase][1])

    def wait_phase(phase):
        for l in range(GROUP):
            pltpu.make_async_copy(utT_hbm.at[:, pl.ds(0, 128)],
                                  uslab_v.at[phase, l],
                                  sems[phase][0]).wait()
            pltpu.make_async_copy(itT_hbm.at[:, pl.ds(0, 128)],
                                  islab_v.at[phase, l],
                                  sems[phase][1]).wait()

    def compute(phase, g):
        wait_phase(phase)
        um16 = jnp.bitwise_and(
            plsc.load_gather(uidx_v, [g * GROUP + row_of_lane]), 127)
        im16 = jnp.bitwise_and(
            plsc.load_gather(iidx_v, [g * GROUP + row_of_lane]), 127)
        acc = jnp.zeros((16,), jnp.float32)
        for j in range(EMBED_DIM // 8):
            dv = lax.shift_left(qblk, 3) + j
            uu = plsc.load_gather(uslab_v.at[phase], [row_of_lane, dv, um16])
            ii = plsc.load_gather(islab_v.at[phase], [row_of_lane, dv, im16])
            acc = acc + uu * ii
        plsc.addupdate_scatter(out_v, [g * GROUP + row_of_lane], acc)

    def body(p, carry):
        for k in range(3):
            g = p * 3 + k
            compute(k, g)
            issue(k, jnp.minimum(g + 3, GROUPS - 1))
        return carry

    issue(0, 0)
    issue(1, 1)
    issue(2, 2)
    lax.fori_loop(0, (GROUPS - 1) // 3, body, 0)
    compute(0, GROUPS - 1)
    wait_phase(1)
    wait_phase(2)

    pltpu.sync_copy(out_v, out_hbm.at[pl.ds(base, B_PER_W)])


def kernel(inputs, user_table, item_table):
    user_ids = inputs[:, 0].astype(jnp.int32)
    item_ids = inputs[:, 1].astype(jnp.int32)
    return _dot_kernel(user_ids, item_ids, user_table.T, item_table.T)
